# baseline (speedup 1.0000x reference)
import functools

import jax
import jax.numpy as jnp
from jax import lax
from jax.experimental import pallas as pl
from jax.experimental.pallas import tpu as pltpu
from jax.experimental.pallas import tpu_sc as plsc

N = 10000
E = 320000
D = 128
H = 256
OUT = 128
NE = 3
NIN = D * (NE + 1)

TROWS = NE * N
QROWS = 7680
GARB = 128
ACC_ROWS = QROWS + GARB
NSUB = 16
ZPS = ACC_ROWS // NSUB
DPS = QROWS // NSUB
OROWS = 4 * QROWS

EDGES_PER_SUB = E // NSUB
CH = 128
NFULL = EDGES_PER_SUB // CH
TAIL = EDGES_PER_SUB - NFULL * CH

RB = 400
NBLK = N // RB


def _edge_mlp_body(nf_ref, w1_ref, b1_ref, w2_ref, b2_ref, t_ref):
    x = nf_ref[...]
    h = jnp.maximum(
        jnp.dot(x, w1_ref[0], preferred_element_type=jnp.float32) + b1_ref[0],
        0.0)
    t_ref[...] = jnp.maximum(
        jnp.dot(h, w2_ref[0], preferred_element_type=jnp.float32) + b2_ref[0],
        0.0)


def _edge_mlps(nf, ew1, eb1, ew2, eb2):
    return pl.pallas_call(
        _edge_mlp_body,
        grid=(NE, NBLK),
        in_specs=[
            pl.BlockSpec((RB, D), lambda e, i: (i, 0)),
            pl.BlockSpec((1, D, H), lambda e, i: (e, 0, 0)),
            pl.BlockSpec((1, 1, H), lambda e, i: (e, 0, 0)),
            pl.BlockSpec((1, H, OUT), lambda e, i: (e, 0, 0)),
            pl.BlockSpec((1, 1, OUT), lambda e, i: (e, 0, 0)),
        ],
        out_specs=pl.BlockSpec((RB, OUT), lambda e, i: (e * NBLK + i, 0)),
        out_shape=jax.ShapeDtypeStruct((TROWS, OUT), jnp.float32),
    )(nf, ew1, eb1, ew2, eb2)


@functools.lru_cache(maxsize=1)
def _make_sc_agg():
    mesh = plsc.VectorSubcoreMesh(core_axis_name="c", subcore_axis_name="s")

    @functools.partial(
        pl.kernel,
        mesh=mesh,
        out_type=jax.ShapeDtypeStruct((OROWS, OUT), jnp.float32),
        scratch_types=[
            pltpu.VMEM((CH,), jnp.int32),
            pltpu.VMEM((CH,), jnp.int32),
            pltpu.VMEM((CH,), jnp.int32),
            pltpu.VMEM((CH,), jnp.int32),
            pltpu.VMEM((CH,), jnp.int32),
            pltpu.VMEM((CH, OUT), jnp.float32),
            pltpu.VMEM((TAIL,), jnp.int32),
            pltpu.VMEM((TAIL,), jnp.int32),
            pltpu.VMEM((TAIL,), jnp.int32),
            pltpu.VMEM((TAIL,), jnp.int32),
            pltpu.VMEM((TAIL,), jnp.int32),
            pltpu.VMEM((TAIL, OUT), jnp.float32),
            pltpu.VMEM_SHARED((ACC_ROWS, OUT), jnp.float32),
            pltpu.SemaphoreType.DMA,
        ],
    )
    def sc_agg(t_hbm, src_hbm, dst_hbm, et_hbm, z_hbm, out_hbm,
               etb, srcb, dstb, gidx, oidx, rows,
               etb2, srcb2, dstb2, gidx2, oidx2, rows2,
               acc, sem):
        c = lax.axis_index("c")
        s = lax.axis_index("s")
        base = s * EDGES_PER_SUB

        def do_chunk(lo, off, k, etb, srcb, dstb, gidx, oidx, rows):
            pltpu.sync_copy(et_hbm.at[pl.ds(off, k)], etb)
            pltpu.sync_copy(src_hbm.at[pl.ds(off, k)], srcb)
            pltpu.sync_copy(dst_hbm.at[pl.ds(off, k)], dstb)

            @pl.loop(0, k, step=16)
            def _(j):
                et = etb[pl.ds(j, 16)]
                flat = et * N + dstb[pl.ds(j, 16)]
                local = flat - lo
                in_range = (local >= 0) & (local < QROWS)
                gidx[pl.ds(j, 16)] = jnp.where(
                    in_range, et * N + srcb[pl.ds(j, 16)], 0)
                oidx[pl.ds(j, 16)] = jnp.where(
                    in_range, local, QROWS + (flat & (GARB - 1)))

            pltpu.async_copy(t_hbm.at[gidx], rows, sem).wait()
            pltpu.sync_copy(rows, acc.at[oidx], add=True)

        for p in range(2):
            q = 2 * c + p
            lo = q * QROWS
            pltpu.sync_copy(z_hbm, acc.at[pl.ds(s * ZPS, ZPS)])
            plsc.subcore_barrier()

            @pl.loop(0, NFULL)
            def _(i):
                do_chunk(lo, base + i * CH, CH,
                         etb, srcb, dstb, gidx, oidx, rows)

            do_chunk(lo, base + NFULL * CH, TAIL,
                     etb2, srcb2, dstb2, gidx2, oidx2, rows2)

            plsc.subcore_barrier()
            pltpu.sync_copy(
                acc.at[pl.ds(s * DPS, DPS)],
                out_hbm.at[pl.ds(lo + s * DPS, DPS)])
            plsc.subcore_barrier()

    return sc_agg


def _node_mlp_body(nf_ref, a0_ref, a1_ref, a2_ref, sel_ref,
                   w1_ref, b1_ref, w2_ref, b2_ref, o_ref):
    x0 = jnp.maximum(nf_ref[...], 0.0)
    x = jnp.concatenate([x0, a0_ref[...], a1_ref[...], a2_ref[...]], axis=1)
    outs = []
    for t in range(2):
        h = jnp.maximum(
            jnp.dot(x, w1_ref[t], preferred_element_type=jnp.float32)
            + b1_ref[t], 0.0)
        outs.append(
            jnp.dot(h, w2_ref[t], preferred_element_type=jnp.float32)
            + b2_ref[t])
    sel = sel_ref[...]
    o_ref[...] = outs[0] + sel * (outs[1] - outs[0])


def _agg_spec(e):
    return pl.BlockSpec((RB, OUT), lambda i: (e * NBLK + i, 0))


def _node_mlps(nf, agg, sel, nw1, nb1, nw2, nb2):
    return pl.pallas_call(
        _node_mlp_body,
        grid=(NBLK,),
        in_specs=[
            pl.BlockSpec((RB, D), lambda i: (i, 0)),
            _agg_spec(0), _agg_spec(1), _agg_spec(2),
            pl.BlockSpec((RB, 1), lambda i: (i, 0)),
            pl.BlockSpec((2, NIN, H), lambda i: (0, 0, 0)),
            pl.BlockSpec((2, 1, H), lambda i: (0, 0, 0)),
            pl.BlockSpec((2, H, OUT), lambda i: (0, 0, 0)),
            pl.BlockSpec((2, 1, OUT), lambda i: (0, 0, 0)),
        ],
        out_specs=pl.BlockSpec((RB, OUT), lambda i: (i, 0)),
        out_shape=jax.ShapeDtypeStruct((N, OUT), jnp.float32),
    )(nf, agg, agg, agg, sel, nw1, nb1, nw2, nb2)


def kernel(node_feature, edge_index, edge_type, node_type,
           ew1_0, eb1_0, ew2_0, eb2_0,
           ew1_1, eb1_1, ew2_1, eb2_1,
           ew1_2, eb1_2, ew2_2, eb2_2,
           nw1_0, nb1_0, nw2_0, nb2_0,
           nw1_1, nb1_1, nw2_1, nb2_1):
    ew1 = jnp.stack([ew1_0, ew1_1, ew1_2])
    eb1 = jnp.stack([eb1_0, eb1_1, eb1_2])[:, None, :]
    ew2 = jnp.stack([ew2_0, ew2_1, ew2_2])
    eb2 = jnp.stack([eb2_0, eb2_1, eb2_2])[:, None, :]
    nw1 = jnp.stack([nw1_0, nw1_1])
    nb1 = jnp.stack([nb1_0, nb1_1])[:, None, :]
    nw2 = jnp.stack([nw2_0, nw2_1])
    nb2 = jnp.stack([nb2_0, nb2_1])[:, None, :]

    t = _edge_mlps(node_feature, ew1, eb1, ew2, eb2)
    zeros = jnp.zeros((ZPS, OUT), jnp.float32)
    agg = _make_sc_agg()(t, edge_index[0], edge_index[1], edge_type, zeros)

    sel = node_type.astype(jnp.float32)[:, None]
    return _node_mlps(node_feature, agg, sel, nw1, nb1, nw2, nb2)


# speedup vs baseline: 30.4957x; 30.4957x over previous
import functools

import jax
import jax.numpy as jnp
from jax import lax
from jax.experimental import pallas as pl
from jax.experimental.pallas import tpu as pltpu
from jax.experimental.pallas import tpu_sc as plsc

N = 10000
E = 320000
D = 128
H = 256
OUT = 128
NE = 3
NIN = D * (NE + 1)

TROWS = NE * N
QROWS = 7680
GARB = 128
ACC_ROWS = QROWS + GARB
NSUB = 16
ZPS = ACC_ROWS // NSUB
DPS = QROWS // NSUB
OROWS = 4 * QROWS

EDGES_PER_SUB = E // NSUB
CH = 128
NFULL = EDGES_PER_SUB // CH
TAIL = EDGES_PER_SUB - NFULL * CH

RB = 400
NBLK = N // RB


def _edge_mlp_body(nf_ref, w1_ref, b1_ref, w2_ref, b2_ref, t_ref):
    x = nf_ref[...]
    h = jnp.maximum(
        jnp.dot(x, w1_ref[0], preferred_element_type=jnp.float32) + b1_ref[0],
        0.0)
    t_ref[...] = jnp.maximum(
        jnp.dot(h, w2_ref[0], preferred_element_type=jnp.float32) + b2_ref[0],
        0.0)


def _edge_mlps(nf, ew1, eb1, ew2, eb2):
    return pl.pallas_call(
        _edge_mlp_body,
        grid=(NE, NBLK),
        in_specs=[
            pl.BlockSpec((RB, D), lambda e, i: (i, 0)),
            pl.BlockSpec((1, D, H), lambda e, i: (e, 0, 0)),
            pl.BlockSpec((1, 1, H), lambda e, i: (e, 0, 0)),
            pl.BlockSpec((1, H, OUT), lambda e, i: (e, 0, 0)),
            pl.BlockSpec((1, 1, OUT), lambda e, i: (e, 0, 0)),
        ],
        out_specs=pl.BlockSpec((RB, OUT), lambda e, i: (e * NBLK + i, 0)),
        out_shape=jax.ShapeDtypeStruct((TROWS, OUT), jnp.float32),
    )(nf, ew1, eb1, ew2, eb2)


@functools.lru_cache(maxsize=1)
def _make_sc_agg():
    mesh = plsc.VectorSubcoreMesh(core_axis_name="c", subcore_axis_name="s")

    @functools.partial(
        pl.kernel,
        mesh=mesh,
        out_type=jax.ShapeDtypeStruct((OROWS, OUT), jnp.float32),
        scratch_types=[
            pltpu.VMEM((CH,), jnp.int32),
            pltpu.VMEM((CH,), jnp.int32),
            pltpu.VMEM((CH,), jnp.int32),
            pltpu.VMEM((CH,), jnp.int32),
            pltpu.VMEM((CH,), jnp.int32),
            pltpu.VMEM((CH, OUT), jnp.float32),
            pltpu.VMEM((TAIL,), jnp.int32),
            pltpu.VMEM((TAIL,), jnp.int32),
            pltpu.VMEM((TAIL,), jnp.int32),
            pltpu.VMEM((TAIL,), jnp.int32),
            pltpu.VMEM((TAIL,), jnp.int32),
            pltpu.VMEM((TAIL, OUT), jnp.float32),
            pltpu.VMEM_SHARED((ACC_ROWS, OUT), jnp.float32),
            pltpu.SemaphoreType.DMA,
        ],
    )
    def sc_agg(t_hbm, src_hbm, dst_hbm, et_hbm, z_hbm, out_hbm,
               etb, srcb, dstb, gidx, oidx, rows,
               etb2, srcb2, dstb2, gidx2, oidx2, rows2,
               acc, sem):
        c = lax.axis_index("c")
        s = lax.axis_index("s")
        base = s * EDGES_PER_SUB

        def do_chunk(lo, off, k, etb, srcb, dstb, gidx, oidx, rows):
            pltpu.sync_copy(et_hbm.at[pl.ds(off, k)], etb)
            pltpu.sync_copy(src_hbm.at[pl.ds(off, k)], srcb)
            pltpu.sync_copy(dst_hbm.at[pl.ds(off, k)], dstb)

            @pl.loop(0, k, step=16)
            def _(j):
                et = etb[pl.ds(j, 16)]
                flat = et * N + dstb[pl.ds(j, 16)]
                local = flat - lo
                in_range = (local >= 0) & (local < QROWS)
                gidx[pl.ds(j, 16)] = et * N + srcb[pl.ds(j, 16)]
                oidx[pl.ds(j, 16)] = jnp.where(
                    in_range, local, QROWS + (flat & (GARB - 1)))

            pltpu.async_copy(t_hbm.at[gidx], rows, sem).wait()
            pltpu.sync_copy(rows, acc.at[oidx], add=True)

        for p in range(2):
            q = 2 * c + p
            lo = q * QROWS
            pltpu.sync_copy(z_hbm, acc.at[pl.ds(s * ZPS, ZPS)])
            plsc.subcore_barrier()

            @pl.loop(0, NFULL)
            def _(i):
                do_chunk(lo, base + i * CH, CH,
                         etb, srcb, dstb, gidx, oidx, rows)

            do_chunk(lo, base + NFULL * CH, TAIL,
                     etb2, srcb2, dstb2, gidx2, oidx2, rows2)

            plsc.subcore_barrier()
            pltpu.sync_copy(
                acc.at[pl.ds(s * DPS, DPS)],
                out_hbm.at[pl.ds(lo + s * DPS, DPS)])
            plsc.subcore_barrier()

    return sc_agg


def _node_mlp_body(nf_ref, a0_ref, a1_ref, a2_ref, sel_ref,
                   w1_ref, b1_ref, w2_ref, b2_ref, o_ref):
    x0 = jnp.maximum(nf_ref[...], 0.0)
    x = jnp.concatenate([x0, a0_ref[...], a1_ref[...], a2_ref[...]], axis=1)
    outs = []
    for t in range(2):
        h = jnp.maximum(
            jnp.dot(x, w1_ref[t], preferred_element_type=jnp.float32)
            + b1_ref[t], 0.0)
        outs.append(
            jnp.dot(h, w2_ref[t], preferred_element_type=jnp.float32)
            + b2_ref[t])
    sel = sel_ref[...]
    o_ref[...] = outs[0] + sel * (outs[1] - outs[0])


def _agg_spec(e):
    return pl.BlockSpec((RB, OUT), lambda i: (e * NBLK + i, 0))


def _node_mlps(nf, agg, sel, nw1, nb1, nw2, nb2):
    return pl.pallas_call(
        _node_mlp_body,
        grid=(NBLK,),
        in_specs=[
            pl.BlockSpec((RB, D), lambda i: (i, 0)),
            _agg_spec(0), _agg_spec(1), _agg_spec(2),
            pl.BlockSpec((RB, 1), lambda i: (i, 0)),
            pl.BlockSpec((2, NIN, H), lambda i: (0, 0, 0)),
            pl.BlockSpec((2, 1, H), lambda i: (0, 0, 0)),
            pl.BlockSpec((2, H, OUT), lambda i: (0, 0, 0)),
            pl.BlockSpec((2, 1, OUT), lambda i: (0, 0, 0)),
        ],
        out_specs=pl.BlockSpec((RB, OUT), lambda i: (i, 0)),
        out_shape=jax.ShapeDtypeStruct((N, OUT), jnp.float32),
    )(nf, agg, agg, agg, sel, nw1, nb1, nw2, nb2)


def kernel(node_feature, edge_index, edge_type, node_type,
           ew1_0, eb1_0, ew2_0, eb2_0,
           ew1_1, eb1_1, ew2_1, eb2_1,
           ew1_2, eb1_2, ew2_2, eb2_2,
           nw1_0, nb1_0, nw2_0, nb2_0,
           nw1_1, nb1_1, nw2_1, nb2_1):
    ew1 = jnp.stack([ew1_0, ew1_1, ew1_2])
    eb1 = jnp.stack([eb1_0, eb1_1, eb1_2])[:, None, :]
    ew2 = jnp.stack([ew2_0, ew2_1, ew2_2])
    eb2 = jnp.stack([eb2_0, eb2_1, eb2_2])[:, None, :]
    nw1 = jnp.stack([nw1_0, nw1_1])
    nb1 = jnp.stack([nb1_0, nb1_1])[:, None, :]
    nw2 = jnp.stack([nw2_0, nw2_1])
    nb2 = jnp.stack([nb2_0, nb2_1])[:, None, :]

    t = _edge_mlps(node_feature, ew1, eb1, ew2, eb2)
    zeros = jnp.zeros((ZPS, OUT), jnp.float32)
    agg = _make_sc_agg()(t, edge_index[0], edge_index[1], edge_type, zeros)

    sel = node_type.astype(jnp.float32)[:, None]
    return _node_mlps(node_feature, agg, sel, nw1, nb1, nw2, nb2)
